# Initial kernel scaffold; baseline (speedup 1.0000x reference)
import jax
import jax.numpy as jnp
from jax.experimental import pallas as pl


def kernel(x, Wq, Wk, Wv, Wo):
    raise NotImplementedError("write your pallas kernel here")


# speedup vs baseline: 1.0284x; 1.0284x over previous
import functools

import jax
import jax.numpy as jnp
import numpy as np
from jax.experimental import pallas as pl

D_MODEL = 4096
N_HEADS = 32
N_KV_HEADS = 8
HEAD_DIM = 128
GROUP = N_HEADS // N_KV_HEADS
B = 32
T = 4
NTOK = B * T


def _attn_group_kernel(x_ref, wq_ref, wk_ref, wv_ref, wo_ref, out_ref):
    g = pl.program_id(0)
    x = x_ref[...]

    k = jnp.dot(x, wk_ref[...], preferred_element_type=jnp.float32)
    v = jnp.dot(x, wv_ref[...], preferred_element_type=jnp.float32)

    row = jax.lax.broadcasted_iota(jnp.int32, (NTOK, NTOK), 0)
    col = jax.lax.broadcasted_iota(jnp.int32, (NTOK, NTOK), 1)
    valid = (col <= row) & (col >= (row // T) * T)

    scale = jnp.float32(1.0 / np.sqrt(HEAD_DIM))
    acc = jnp.zeros((NTOK, D_MODEL), jnp.float32)
    for h in range(GROUP):
        qh = jnp.dot(
            x,
            wq_ref[:, h * HEAD_DIM:(h + 1) * HEAD_DIM],
            preferred_element_type=jnp.float32,
        )
        s = jax.lax.dot_general(
            qh, k, (((1,), (1,)), ((), ())),
            preferred_element_type=jnp.float32,
        ) * scale
        s = jnp.where(valid, s, jnp.float32(-1e30))
        m = jnp.max(s, axis=1, keepdims=True)
        p = jnp.exp(s - m)
        p = p / jnp.sum(p, axis=1, keepdims=True)
        oh = jnp.dot(p, v, preferred_element_type=jnp.float32)
        acc += jnp.dot(
            oh,
            wo_ref[h * HEAD_DIM:(h + 1) * HEAD_DIM, :],
            preferred_element_type=jnp.float32,
        )

    @pl.when(g == 0)
    def _init():
        out_ref[...] = acc

    @pl.when(g > 0)
    def _accum():
        out_ref[...] += acc


@jax.jit
def kernel(x, Wq, Wk, Wv, Wo):
    Bx, Tx, Dx = x.shape
    xf = x.reshape(Bx * Tx, Dx)
    out = pl.pallas_call(
        _attn_group_kernel,
        grid=(N_KV_HEADS,),
        in_specs=[
            pl.BlockSpec((NTOK, D_MODEL), lambda g: (0, 0)),
            pl.BlockSpec((D_MODEL, GROUP * HEAD_DIM), lambda g: (0, g)),
            pl.BlockSpec((D_MODEL, HEAD_DIM), lambda g: (0, g)),
            pl.BlockSpec((D_MODEL, HEAD_DIM), lambda g: (0, g)),
            pl.BlockSpec((GROUP * HEAD_DIM, D_MODEL), lambda g: (g, 0)),
        ],
        out_specs=pl.BlockSpec((NTOK, D_MODEL), lambda g: (0, 0)),
        out_shape=jax.ShapeDtypeStruct((NTOK, D_MODEL), jnp.float32),
    )(xf, Wq, Wk, Wv, Wo)
    return out.reshape(Bx, Tx, Dx)
